# Initial kernel scaffold; baseline (speedup 1.0000x reference)
import jax
import jax.numpy as jnp
from jax.experimental import pallas as pl


def kernel(x, table, W, b, a):
    raise NotImplementedError("write your pallas kernel here")


# speedup vs baseline: 1.6253x; 1.6253x over previous
import functools

import jax
import jax.numpy as jnp
from jax import lax
from jax.experimental import pallas as pl
from jax.experimental.pallas import tpu as pltpu
from jax.experimental.pallas import tpu_sc as plsc

NUM_CLASSES = 1000000
EMBED_DIM = 64
BATCH = 16384
HIST = 50

B_TOTAL = BATCH * HIST
NW = 32
ROWS_W = B_TOTAL // NW
CHUNK = 128
NCH_W = ROWS_W // CHUNK
NBUF = 2


def _sc_gather(table, idx3):
    mesh = plsc.VectorSubcoreMesh(core_axis_name="c", subcore_axis_name="s")

    @functools.partial(
        pl.kernel,
        mesh=mesh,
        compiler_params=pltpu.CompilerParams(use_tc_tiling_on_sc=False),
        out_type=jax.ShapeDtypeStruct((B_TOTAL, EMBED_DIM), jnp.float32),
        scratch_types=[
            pltpu.VMEM((NCH_W, CHUNK), jnp.int32),
            pltpu.VMEM((NBUF, CHUNK, EMBED_DIM), jnp.float32),
            pltpu.SemaphoreType.DMA,
            pltpu.SemaphoreType.DMA,
        ],
    )
    def k(table_hbm, idx_hbm, out_hbm, idx_v, rows_v, sem0, sem1):
        wid = lax.axis_index("s") * 2 + lax.axis_index("c")
        base = wid * ROWS_W
        sems = (sem0, sem1)

        pltpu.sync_copy(idx_hbm.at[wid], idx_v)

        def start(ch, b):
            pltpu.async_copy(table_hbm.at[idx_v.at[ch]], rows_v.at[b], sems[b])

        def wait(ch, b):
            pltpu.make_async_copy(
                table_hbm.at[idx_v.at[ch]], rows_v.at[b], sems[b]
            ).wait()

        for b in range(NBUF):
            start(b, b)

        def body(g, _):
            for b in range(NBUF):
                ch = g * NBUF + b
                wait(ch, b)
                pltpu.sync_copy(
                    rows_v.at[b], out_hbm.at[pl.ds(base + ch * CHUNK, CHUNK)]
                )
                nxt = ch + NBUF

                @pl.when(nxt < NCH_W)
                def _():
                    start(nxt, b)

            return _

        lax.fori_loop(0, NCH_W // NBUF, body, None)

    return k(table, idx3)


def _tc_linear_prelu(e2, w2, b2, a):
    M = e2.shape[0]
    BLK = 8192
    grid = (M // BLK,)

    def body(a_ref, e_ref, w_ref, b_ref, o_ref):
        y = jnp.dot(e_ref[...], w_ref[...], preferred_element_type=jnp.float32)
        y = y + b_ref[...]
        o_ref[...] = jnp.where(y >= 0.0, y, a_ref[0] * y)

    return pl.pallas_call(
        body,
        grid=grid,
        in_specs=[
            pl.BlockSpec(memory_space=pltpu.SMEM),
            pl.BlockSpec((BLK, 128), lambda i: (i, 0)),
            pl.BlockSpec((128, 128), lambda i: (0, 0)),
            pl.BlockSpec((1, 128), lambda i: (0, 0)),
        ],
        out_specs=pl.BlockSpec((BLK, 128), lambda i: (i, 0)),
        out_shape=jax.ShapeDtypeStruct((M, 128), jnp.float32),
    )(a, e2, w2, b2)


def kernel(x, table, W, b, a):
    idx3 = x.astype(jnp.int32).reshape(NW, NCH_W, CHUNK)
    e = _sc_gather(table, idx3)
    e2 = e.reshape(B_TOTAL // 2, 2 * EMBED_DIM)
    wt = W.T
    w2 = jnp.kron(jnp.eye(2, dtype=W.dtype), wt)
    b2 = jnp.tile(b, 2).reshape(1, 2 * EMBED_DIM)
    y2 = _tc_linear_prelu(e2, w2, b2, a)
    return y2.reshape(BATCH, HIST, EMBED_DIM)


# speedup vs baseline: 1.6282x; 1.0018x over previous
import functools

import jax
import jax.numpy as jnp
from jax import lax
from jax.experimental import pallas as pl
from jax.experimental.pallas import tpu as pltpu
from jax.experimental.pallas import tpu_sc as plsc

NUM_CLASSES = 1000000
EMBED_DIM = 64
BATCH = 16384
HIST = 50

B_TOTAL = BATCH * HIST
NW = 32
ROWS_W = B_TOTAL // NW
CHUNK = 128
NCH_W = ROWS_W // CHUNK
NBUF = 2

BLK0 = 8192
BLKB = 4096


def _tc_untranspose(table_t):
    n = table_t.shape[1]
    grid = (pl.cdiv(n, BLK0),)

    def body(t_ref, o_ref):
        eye = jnp.eye(EMBED_DIM, dtype=jnp.float32)
        o_ref[...] = lax.dot_general(
            t_ref[...], eye, (((0,), (0,)), ((), ())),
            preferred_element_type=jnp.float32,
        )

    return pl.pallas_call(
        body,
        grid=grid,
        in_specs=[pl.BlockSpec((EMBED_DIM, BLK0), lambda i: (0, i))],
        out_specs=pl.BlockSpec((BLK0, EMBED_DIM), lambda i: (i, 0)),
        out_shape=jax.ShapeDtypeStruct((n, EMBED_DIM), jnp.float32),
    )(table_t)


def _sc_gather(table, idx3):
    mesh = plsc.VectorSubcoreMesh(core_axis_name="c", subcore_axis_name="s")

    @functools.partial(
        pl.kernel,
        mesh=mesh,
        compiler_params=pltpu.CompilerParams(use_tc_tiling_on_sc=False),
        out_type=jax.ShapeDtypeStruct((B_TOTAL, EMBED_DIM), jnp.float32),
        scratch_types=[
            pltpu.VMEM((NCH_W, CHUNK), jnp.int32),
            pltpu.VMEM((NBUF, CHUNK, EMBED_DIM), jnp.float32),
            pltpu.SemaphoreType.DMA,
            pltpu.SemaphoreType.DMA,
        ],
    )
    def k(table_hbm, idx_hbm, out_hbm, idx_v, rows_v, sem0, sem1):
        wid = lax.axis_index("s") * 2 + lax.axis_index("c")
        base = wid * ROWS_W
        sems = (sem0, sem1)

        pltpu.sync_copy(idx_hbm.at[wid], idx_v)

        def start(ch, b):
            pltpu.async_copy(table_hbm.at[idx_v.at[ch]], rows_v.at[b], sems[b])

        def wait(ch, b):
            pltpu.make_async_copy(
                table_hbm.at[idx_v.at[ch]], rows_v.at[b], sems[b]
            ).wait()

        for b in range(NBUF):
            start(b, b)

        def body(g, _):
            for b in range(NBUF):
                ch = g * NBUF + b
                wait(ch, b)
                pltpu.sync_copy(
                    rows_v.at[b], out_hbm.at[pl.ds(base + ch * CHUNK, CHUNK)]
                )
                nxt = ch + NBUF

                @pl.when(nxt < NCH_W)
                def _():
                    start(nxt, b)

            return _

        lax.fori_loop(0, NCH_W // NBUF, body, None)

    return k(table, idx3)


def _tc_linear_prelu_t(g3, w, b2, a):
    def body(a_ref, g_ref, w_ref, b_ref, o_ref):
        e = g_ref[0]
        y = lax.dot_general(
            w_ref[...], e, (((1,), (1,)), ((), ())),
            preferred_element_type=jnp.float32,
        )
        y = y + b_ref[...]
        o_ref[0] = jnp.where(y >= 0.0, y, a_ref[0] * y)

    return pl.pallas_call(
        body,
        grid=(HIST, BATCH // BLKB),
        in_specs=[
            pl.BlockSpec(memory_space=pltpu.SMEM),
            pl.BlockSpec((1, BLKB, EMBED_DIM), lambda l, i: (l, i, 0)),
            pl.BlockSpec((EMBED_DIM, EMBED_DIM), lambda l, i: (0, 0)),
            pl.BlockSpec((EMBED_DIM, 1), lambda l, i: (0, 0)),
        ],
        out_specs=pl.BlockSpec((1, EMBED_DIM, BLKB), lambda l, i: (l, 0, i)),
        out_shape=jax.ShapeDtypeStruct((HIST, EMBED_DIM, BATCH), jnp.float32),
    )(a, g3, w, b2)


def kernel(x, table, W, b, a):
    table_t = jnp.transpose(table)
    idx3 = jnp.transpose(x).astype(jnp.int32).reshape(NW, NCH_W, CHUNK)

    table_rm = _tc_untranspose(table_t)
    g = _sc_gather(table_rm, idx3)
    g3 = g.reshape(HIST, BATCH, EMBED_DIM)
    y_t = _tc_linear_prelu_t(g3, W, b.reshape(EMBED_DIM, 1), a)
    return jnp.transpose(y_t, (2, 0, 1))


# speedup vs baseline: 3.5424x; 2.1757x over previous
import functools

import jax
import jax.numpy as jnp
from jax import lax
from jax.experimental import pallas as pl
from jax.experimental.pallas import tpu as pltpu
from jax.experimental.pallas import tpu_sc as plsc

NUM_CLASSES = 1000000
EMBED_DIM = 64
BATCH = 16384
HIST = 50

B_TOTAL = BATCH * HIST
NW = 32
ROWS_W = B_TOTAL // NW
CHUNK = 128
NCH_W = ROWS_W // CHUNK
NBUF = 2

BLKH = 8192
NG0 = 62
N_PACKED = NG0 * BLKH
BLKB = 4096
HALF = BLKB // 2


def _tc_pack_table(table_t):
    def body(ta_ref, tb_ref, o_ref):
        eye = jnp.eye(EMBED_DIM, dtype=jnp.float32)
        dn = (((0,), (0,)), ((), ()))
        o_ref[:, 0:EMBED_DIM] = lax.dot_general(
            ta_ref[...], eye, dn, preferred_element_type=jnp.float32)
        o_ref[:, EMBED_DIM:2 * EMBED_DIM] = lax.dot_general(
            tb_ref[...], eye, dn, preferred_element_type=jnp.float32)

    return pl.pallas_call(
        body,
        grid=(NG0,),
        in_specs=[
            pl.BlockSpec((EMBED_DIM, BLKH), lambda g: (0, 2 * g)),
            pl.BlockSpec((EMBED_DIM, BLKH),
                         lambda g: (0, jnp.minimum(2 * g + 1, 2 * NG0 - 3))),
        ],
        out_specs=pl.BlockSpec((BLKH, 2 * EMBED_DIM), lambda g: (g, 0)),
        out_shape=jax.ShapeDtypeStruct((N_PACKED, 2 * EMBED_DIM), jnp.float32),
    )(table_t, table_t)


def _sc_gather(table64, idx2):
    mesh = plsc.VectorSubcoreMesh(core_axis_name="c", subcore_axis_name="s")

    @functools.partial(
        pl.kernel,
        mesh=mesh,
        compiler_params=pltpu.CompilerParams(use_tc_tiling_on_sc=False),
        out_type=jax.ShapeDtypeStruct((B_TOTAL // 2, 2 * EMBED_DIM),
                                      jnp.float32),
        scratch_types=[
            pltpu.VMEM((ROWS_W,), jnp.int32),
            pltpu.VMEM((NBUF, CHUNK, EMBED_DIM), jnp.float32),
            pltpu.SemaphoreType.DMA,
            pltpu.SemaphoreType.DMA,
        ],
    )
    def k(table_hbm, idx_hbm, out_hbm, idx_v, rows_v, sem0, sem1):
        wid = lax.axis_index("s") * 2 + lax.axis_index("c")
        base = wid * ROWS_W
        sems = (sem0, sem1)

        pltpu.sync_copy(idx_hbm.at[wid], idx_v)

        def remap(kk, _):
            v = idx_v[pl.ds(kk * 16, 16)]
            vp = (v & -16384) | ((v & 8191) << 1) | ((v >> 13) & 1)
            idx_v[pl.ds(kk * 16, 16)] = vp
            return _

        lax.fori_loop(0, ROWS_W // 16, remap, None)

        def start(ch, b):
            pltpu.async_copy(
                table_hbm.at[idx_v.at[pl.ds(ch * CHUNK, CHUNK)]],
                rows_v.at[b], sems[b])

        def wait(ch, b):
            pltpu.make_async_copy(
                table_hbm.at[idx_v.at[pl.ds(ch * CHUNK, CHUNK)]],
                rows_v.at[b], sems[b]).wait()

        def store(ch, b):
            p0 = base + ch * CHUNK
            l = p0 >> 14
            b0 = p0 & 16383
            r0 = pl.multiple_of((l << 13) + ((b0 >> 12) << 11) + (b0 & 2047),
                                CHUNK)
            col = pl.multiple_of(((b0 >> 11) & 1) << 6, EMBED_DIM)
            pltpu.sync_copy(
                rows_v.at[b],
                out_hbm.at[pl.ds(r0, CHUNK), pl.ds(col, EMBED_DIM)])

        for b in range(NBUF):
            start(b, b)

        def body(g, _):
            for b in range(NBUF):
                ch = g * NBUF + b
                wait(ch, b)
                store(ch, b)
                nxt = ch + NBUF

                @pl.when(nxt < NCH_W)
                def _():
                    start(nxt, b)

            return _

        lax.fori_loop(0, NCH_W // NBUF, body, None)

    return k(table64, idx2)


def _tc_linear_prelu_t(p2, w, b2, a):
    nbb = BATCH // BLKB

    def body(a_ref, p_ref, w_ref, b_ref, o_ref):
        dn = (((1,), (1,)), ((), ()))
        alpha = a_ref[0]
        for h in range(2):
            e = p_ref[:, h * EMBED_DIM:(h + 1) * EMBED_DIM]
            y = lax.dot_general(w_ref[...], e, dn,
                                preferred_element_type=jnp.float32)
            y = y + b_ref[...]
            o_ref[0, :, h * HALF:(h + 1) * HALF] = jnp.where(
                y >= 0.0, y, alpha * y)

    return pl.pallas_call(
        body,
        grid=(HIST, nbb),
        in_specs=[
            pl.BlockSpec(memory_space=pltpu.SMEM),
            pl.BlockSpec((HALF, 2 * EMBED_DIM), lambda l, i: (l * nbb + i, 0)),
            pl.BlockSpec((EMBED_DIM, EMBED_DIM), lambda l, i: (0, 0)),
            pl.BlockSpec((EMBED_DIM, 1), lambda l, i: (0, 0)),
        ],
        out_specs=pl.BlockSpec((1, EMBED_DIM, BLKB), lambda l, i: (l, 0, i)),
        out_shape=jax.ShapeDtypeStruct((HIST, EMBED_DIM, BATCH), jnp.float32),
    )(a, p2, w, b2)


def kernel(x, table, W, b, a):
    table_t = jnp.transpose(table)
    idx2 = jnp.transpose(x).astype(jnp.int32).reshape(NW, ROWS_W)

    table_pk = _tc_pack_table(table_t)
    table64 = table_pk.reshape(2 * N_PACKED, EMBED_DIM)
    p2 = _sc_gather(table64, idx2)
    y_t = _tc_linear_prelu_t(p2, W, b.reshape(EMBED_DIM, 1), a)
    return jnp.transpose(y_t, (2, 0, 1))


# speedup vs baseline: 4.1502x; 1.1716x over previous
import functools

import jax
import jax.numpy as jnp
from jax import lax
from jax.experimental import pallas as pl
from jax.experimental.pallas import tpu as pltpu
from jax.experimental.pallas import tpu_sc as plsc

NUM_CLASSES = 1000000
EMBED_DIM = 64
BATCH = 16384
HIST = 50

B_TOTAL = BATCH * HIST
NW = 32
ROWS_W = B_TOTAL // NW
CHUNK = 128
NCH_W = ROWS_W // CHUNK
NBUF = 8
AHEAD = 4

BLKH = 16384
NG0 = 31
N_PACKED = NG0 * BLKH
BLKB = 8192
HALF = BLKB // 2


def _tc_pack_table(table_t):
    def body(ta_ref, tb_ref, o_ref):
        eye = jnp.eye(EMBED_DIM, dtype=jnp.float32)
        dn = (((0,), (0,)), ((), ()))
        o_ref[:, 0:EMBED_DIM] = lax.dot_general(
            ta_ref[...], eye, dn, preferred_element_type=jnp.float32)
        o_ref[:, EMBED_DIM:2 * EMBED_DIM] = lax.dot_general(
            tb_ref[...], eye, dn, preferred_element_type=jnp.float32)

    return pl.pallas_call(
        body,
        grid=(NG0,),
        in_specs=[
            pl.BlockSpec((EMBED_DIM, BLKH), lambda g: (0, 2 * g)),
            pl.BlockSpec((EMBED_DIM, BLKH), lambda g: (0, 2 * g + 1)),
        ],
        out_specs=pl.BlockSpec((BLKH, 2 * EMBED_DIM), lambda g: (g, 0)),
        out_shape=jax.ShapeDtypeStruct((N_PACKED, 2 * EMBED_DIM), jnp.float32),
    )(table_t, table_t)


def _sc_gather(table64, idx2):
    mesh = plsc.VectorSubcoreMesh(core_axis_name="c", subcore_axis_name="s")

    @functools.partial(
        pl.kernel,
        mesh=mesh,
        compiler_params=pltpu.CompilerParams(use_tc_tiling_on_sc=False),
        out_type=jax.ShapeDtypeStruct((B_TOTAL // 2, 2 * EMBED_DIM),
                                      jnp.float32),
        scratch_types=[
            pltpu.VMEM((ROWS_W,), jnp.int32),
            pltpu.VMEM((NBUF, CHUNK, EMBED_DIM), jnp.float32),
        ]
        + [pltpu.SemaphoreType.DMA] * (2 * NBUF),
    )
    def k(table_hbm, idx_hbm, out_hbm, idx_v, rows_v, *sems):
        gsem = sems[:NBUF]
        ssem = sems[NBUF:]
        wid = lax.axis_index("s") * 2 + lax.axis_index("c")
        base = wid * ROWS_W

        pltpu.sync_copy(idx_hbm.at[wid], idx_v)

        def remap(kk, _):
            v = idx_v[pl.ds(kk * 16, 16)]
            vp = (v & -32768) | ((v & 16383) << 1) | ((v >> 14) & 1)
            idx_v[pl.ds(kk * 16, 16)] = vp
            return _

        lax.fori_loop(0, ROWS_W // 16, remap, None)

        def start_g(ch, b):
            pltpu.async_copy(
                table_hbm.at[idx_v.at[pl.ds(ch * CHUNK, CHUNK)]],
                rows_v.at[b], gsem[b])

        def wait_g(ch, b):
            pltpu.make_async_copy(
                table_hbm.at[idx_v.at[pl.ds(ch * CHUNK, CHUNK)]],
                rows_v.at[b], gsem[b]).wait()

        def out_slice(ch):
            p0 = base + ch * CHUNK
            l = p0 >> 14
            b0 = p0 & 16383
            r0 = pl.multiple_of(
                (l << 13) + ((b0 >> 13) << 12) + (b0 & 4095), CHUNK)
            col = pl.multiple_of(((b0 >> 12) & 1) << 6, EMBED_DIM)
            return out_hbm.at[pl.ds(r0, CHUNK), pl.ds(col, EMBED_DIM)]

        def start_s(ch, b):
            pltpu.async_copy(rows_v.at[b], out_slice(ch), ssem[b])

        def wait_s(ch, b):
            pltpu.make_async_copy(rows_v.at[b], out_slice(ch),
                                  ssem[b]).wait()

        for b in range(AHEAD):
            start_g(b, b)

        def body(g, _):
            for b in range(NBUF):
                ch = g * NBUF + b
                wait_g(ch, b)
                start_s(ch, b)
                nxt = ch + AHEAD
                bn = (b + AHEAD) % NBUF

                @pl.when(nxt < NCH_W)
                def _():
                    @pl.when(ch >= AHEAD)
                    def _():
                        wait_s(ch - AHEAD, bn)

                    start_g(nxt, bn)

            return _

        lax.fori_loop(0, NCH_W // NBUF, body, None)

        for j in range(2 * AHEAD):
            ch = NCH_W - 2 * AHEAD + j
            wait_s(ch, ch % NBUF)

    return k(table64, idx2)


def _tc_linear_prelu_t(p2, w, b2, a):
    nbb = BATCH // BLKB

    def body(a_ref, p_ref, w_ref, b_ref, o_ref):
        dn = (((1,), (1,)), ((), ()))
        alpha = a_ref[0]
        for h in range(2):
            e = p_ref[:, h * EMBED_DIM:(h + 1) * EMBED_DIM]
            y = lax.dot_general(w_ref[...], e, dn,
                                preferred_element_type=jnp.float32)
            y = y + b_ref[...]
            o_ref[0, :, h * HALF:(h + 1) * HALF] = jnp.where(
                y >= 0.0, y, alpha * y)

    return pl.pallas_call(
        body,
        grid=(HIST, nbb),
        in_specs=[
            pl.BlockSpec(memory_space=pltpu.SMEM),
            pl.BlockSpec((HALF, 2 * EMBED_DIM), lambda l, i: (l * nbb + i, 0)),
            pl.BlockSpec((EMBED_DIM, EMBED_DIM), lambda l, i: (0, 0)),
            pl.BlockSpec((EMBED_DIM, 1), lambda l, i: (0, 0)),
        ],
        out_specs=pl.BlockSpec((1, EMBED_DIM, BLKB), lambda l, i: (l, 0, i)),
        out_shape=jax.ShapeDtypeStruct((HIST, EMBED_DIM, BATCH), jnp.float32),
    )(a, p2, w, b2)


def kernel(x, table, W, b, a):
    table_t = jnp.transpose(table)
    idx2 = jnp.transpose(x).astype(jnp.int32).reshape(NW, ROWS_W)

    table_pk = _tc_pack_table(table_t)
    table64 = table_pk.reshape(2 * N_PACKED, EMBED_DIM)
    p2 = _sc_gather(table64, idx2)
    y_t = _tc_linear_prelu_t(p2, W, b.reshape(EMBED_DIM, 1), a)
    return jnp.transpose(y_t, (2, 0, 1))


# speedup vs baseline: 4.4093x; 1.0624x over previous
import functools

import jax
import jax.numpy as jnp
from jax import lax
from jax.experimental import pallas as pl
from jax.experimental.pallas import tpu as pltpu
from jax.experimental.pallas import tpu_sc as plsc

NUM_CLASSES = 1000000
EMBED_DIM = 64
BATCH = 16384
HIST = 50

B_TOTAL = BATCH * HIST
NW = 32
ROWS_W = B_TOTAL // NW
CHUNK = 128
NCH_W = ROWS_W // CHUNK
NBUF = 8
AHEAD = 4

BLKH = 16384
NG0 = 31
N_PACKED = NG0 * BLKH
BLKB = 16384
HALF = BLKB // 2


def _tc_pack_table(table_t):
    def body(ta_ref, tb_ref, o_ref):
        o_ref[:, 0:EMBED_DIM] = ta_ref[...].T
        o_ref[:, EMBED_DIM:2 * EMBED_DIM] = tb_ref[...].T

    return pl.pallas_call(
        body,
        grid=(NG0,),
        in_specs=[
            pl.BlockSpec((EMBED_DIM, BLKH), lambda g: (0, 2 * g)),
            pl.BlockSpec((EMBED_DIM, BLKH), lambda g: (0, 2 * g + 1)),
        ],
        out_specs=pl.BlockSpec((BLKH, 2 * EMBED_DIM), lambda g: (g, 0)),
        out_shape=jax.ShapeDtypeStruct((N_PACKED, 2 * EMBED_DIM), jnp.float32),
    )(table_t, table_t)


def _sc_gather(table64, idx2):
    mesh = plsc.VectorSubcoreMesh(core_axis_name="c", subcore_axis_name="s")

    @functools.partial(
        pl.kernel,
        mesh=mesh,
        compiler_params=pltpu.CompilerParams(use_tc_tiling_on_sc=False),
        out_type=jax.ShapeDtypeStruct((B_TOTAL // 2, 2 * EMBED_DIM),
                                      jnp.float32),
        scratch_types=[
            pltpu.VMEM((ROWS_W,), jnp.int32),
            pltpu.VMEM((NBUF, CHUNK, EMBED_DIM), jnp.float32),
        ]
        + [pltpu.SemaphoreType.DMA] * (2 * NBUF),
    )
    def k(table_hbm, idx_hbm, out_hbm, idx_v, rows_v, *sems):
        gsem = sems[:NBUF]
        ssem = sems[NBUF:]
        wid = lax.axis_index("s") * 2 + lax.axis_index("c")
        base = wid * ROWS_W

        pltpu.sync_copy(idx_hbm.at[wid], idx_v)

        def remap(kk, _):
            v = idx_v[pl.ds(kk * 16, 16)]
            vp = (v & -32768) | ((v & 16383) << 1) | ((v >> 14) & 1)
            idx_v[pl.ds(kk * 16, 16)] = vp
            return _

        lax.fori_loop(0, ROWS_W // 16, remap, None)

        def start_g(ch, b):
            pltpu.async_copy(
                table_hbm.at[idx_v.at[pl.ds(ch * CHUNK, CHUNK)]],
                rows_v.at[b], gsem[b])

        def wait_g(ch, b):
            pltpu.make_async_copy(
                table_hbm.at[idx_v.at[pl.ds(ch * CHUNK, CHUNK)]],
                rows_v.at[b], gsem[b]).wait()

        def out_slice(ch):
            p0 = base + ch * CHUNK
            l = p0 >> 14
            b0 = p0 & 16383
            r0 = pl.multiple_of((l << 13) + (b0 & 8191), CHUNK)
            col = pl.multiple_of(((b0 >> 13) & 1) << 6, EMBED_DIM)
            return out_hbm.at[pl.ds(r0, CHUNK), pl.ds(col, EMBED_DIM)]

        def start_s(ch, b):
            pltpu.async_copy(rows_v.at[b], out_slice(ch), ssem[b])

        def wait_s(ch, b):
            pltpu.make_async_copy(rows_v.at[b], out_slice(ch),
                                  ssem[b]).wait()

        for b in range(AHEAD):
            start_g(b, b)

        def body(g, _):
            for b in range(NBUF):
                ch = g * NBUF + b
                wait_g(ch, b)
                start_s(ch, b)
                nxt = ch + AHEAD
                bn = (b + AHEAD) % NBUF

                @pl.when(nxt < NCH_W)
                def _():
                    @pl.when(ch >= AHEAD)
                    def _():
                        wait_s(ch - AHEAD, bn)

                    start_g(nxt, bn)

            return _

        lax.fori_loop(0, NCH_W // NBUF, body, None)

        for j in range(2 * AHEAD):
            ch = NCH_W - 2 * AHEAD + j
            wait_s(ch, ch % NBUF)

    return k(table64, idx2)


def _tc_linear_prelu_t(p2, w, b2, a):
    nbb = BATCH // BLKB

    def body(a_ref, p_ref, w_ref, b_ref, o_ref):
        dn = (((1,), (1,)), ((), ()))
        alpha = a_ref[0]
        for h in range(2):
            e = p_ref[:, h * EMBED_DIM:(h + 1) * EMBED_DIM]
            y = lax.dot_general(w_ref[...], e, dn,
                                preferred_element_type=jnp.float32)
            y = y + b_ref[...]
            o_ref[0, :, h * HALF:(h + 1) * HALF] = jnp.where(
                y >= 0.0, y, alpha * y)

    return pl.pallas_call(
        body,
        grid=(HIST, nbb),
        in_specs=[
            pl.BlockSpec(memory_space=pltpu.SMEM),
            pl.BlockSpec((HALF, 2 * EMBED_DIM), lambda l, i: (l * nbb + i, 0)),
            pl.BlockSpec((EMBED_DIM, EMBED_DIM), lambda l, i: (0, 0)),
            pl.BlockSpec((EMBED_DIM, 1), lambda l, i: (0, 0)),
        ],
        out_specs=pl.BlockSpec((1, EMBED_DIM, BLKB), lambda l, i: (l, 0, i)),
        out_shape=jax.ShapeDtypeStruct((HIST, EMBED_DIM, BATCH), jnp.float32),
    )(a, p2, w, b2)


def kernel(x, table, W, b, a):
    table_t = jnp.transpose(table)
    idx2 = jnp.transpose(x).astype(jnp.int32).reshape(NW, ROWS_W)

    table_pk = _tc_pack_table(table_t)
    table64 = table_pk.reshape(2 * N_PACKED, EMBED_DIM)
    p2 = _sc_gather(table64, idx2)
    y_t = _tc_linear_prelu_t(p2, W, b.reshape(EMBED_DIM, 1), a)
    return jnp.transpose(y_t, (2, 0, 1))
